# baseline (speedup 1.0000x reference)
import functools

import jax
import jax.numpy as jnp
from jax import lax
from jax.experimental import pallas as pl
from jax.experimental.pallas import tpu as pltpu
from jax.experimental.pallas import tpu_sc as plsc

_Q = 0.7
_K = 0.5
_B = 16384
_NCLS = 1000
_NCORES = 2
_NSUB = 16
_NW = _NCORES * _NSUB
_PER_W = _B // _NW
_CHUNK = 128
_NCH = _PER_W // _CHUNK
_CONST = (1.0 - _K ** _Q) / _Q
_RB = 1024
_GRID = _B // _RB


def _sc_gather_w(weight_flat, indexes):
    mesh = plsc.VectorSubcoreMesh(core_axis_name="c", subcore_axis_name="s")

    @functools.partial(
        pl.kernel,
        mesh=mesh,
        out_type=jax.ShapeDtypeStruct((_B,), jnp.float32),
        scratch_types=[
            pltpu.VMEM((_PER_W,), jnp.int32),
            pltpu.VMEM((_PER_W,), jnp.float32),
            pltpu.SemaphoreType.DMA,
        ],
    )
    def gather_kernel(weight_hbm, indexes_hbm, w_out, widx_v, w_v, sem):
        wid = lax.axis_index("c") * _NSUB + lax.axis_index("s")
        base = wid * _PER_W
        pltpu.sync_copy(indexes_hbm.at[pl.ds(base, _PER_W)], widx_v)
        copies = []
        for c in range(_NCH):
            sl = pl.ds(c * _CHUNK, _CHUNK)
            copies.append(pltpu.async_copy(
                weight_hbm.at[widx_v.at[sl]], w_v.at[sl], sem))
        for cp in copies:
            cp.wait()
        pltpu.sync_copy(w_v, w_out.at[pl.ds(base, _PER_W)])

    return gather_kernel(weight_flat, indexes)


def _dense_body(t_ref, w_ref, lg_ref, out_ref):
    i = pl.program_id(0)
    lg = lg_ref[...]
    t = t_ref[...].reshape(_RB, 1)
    w = w_ref[...].reshape(_RB, 1)
    col = lax.broadcasted_iota(jnp.int32, (_RB, _NCLS), 1)
    yg = jnp.sum(jnp.where(col == t, lg, 0.0), axis=1, keepdims=True)
    p = jnp.exp(jnp.log(yg) * _Q)
    part = jnp.sum(((1.0 - p) * (1.0 / _Q) - _CONST) * w) * (1.0 / _B)

    @pl.when(i == 0)
    def _():
        out_ref[0, 0] = 0.0

    out_ref[0, 0] += part


def kernel(logits, targets, indexes, weight):
    idx = indexes.astype(jnp.int32)
    tgt = targets.astype(jnp.int32)
    w = _sc_gather_w(weight.reshape(-1), idx)
    out = pl.pallas_call(
        _dense_body,
        grid=(_GRID,),
        in_specs=[
            pl.BlockSpec((_RB,), lambda i: (i,)),
            pl.BlockSpec((_RB,), lambda i: (i,)),
            pl.BlockSpec((_RB, _NCLS), lambda i: (i, 0)),
        ],
        out_specs=pl.BlockSpec((1, 1), lambda i: (0, 0),
                               memory_space=pltpu.SMEM),
        out_shape=jax.ShapeDtypeStruct((1, 1), jnp.float32),
    )(tgt, w, logits)
    return out[0, 0]


# speedup vs baseline: 1.6693x; 1.6693x over previous
import functools

import jax
import jax.numpy as jnp
from jax import lax
from jax.experimental import pallas as pl
from jax.experimental.pallas import tpu as pltpu
from jax.experimental.pallas import tpu_sc as plsc

_Q = 0.7
_K = 0.5
_B = 16384
_NCLS = 1000
_NCORES = 2
_NSUB = 16
_NW = _NCORES * _NSUB
_PER_W = _B // _NW
_CHUNK = 128
_NCH = _PER_W // _CHUNK
_CONST = (1.0 - _K ** _Q) / _Q


def _sc_gather_blocks(lt, targets):
    mesh = plsc.VectorSubcoreMesh(core_axis_name="c", subcore_axis_name="s")

    @functools.partial(
        pl.kernel,
        mesh=mesh,
        out_type=jax.ShapeDtypeStruct((_B // _CHUNK, _CHUNK, _CHUNK), jnp.float32),
        scratch_types=[
            pltpu.VMEM((_PER_W,), jnp.int32),
            pltpu.VMEM((_CHUNK, _CHUNK), jnp.float32),
            pltpu.VMEM((_CHUNK, _CHUNK), jnp.float32),
            pltpu.SemaphoreType.DMA,
            pltpu.SemaphoreType.DMA,
        ],
    )
    def blocks_kernel(lt_hbm, t_hbm, blk_out, tgt_v, blk_a, blk_b, sem_g, sem_o):
        wid = lax.axis_index("c") * _NSUB + lax.axis_index("s")
        base = wid * _PER_W
        pltpu.sync_copy(t_hbm.at[pl.ds(base, _PER_W)], tgt_v)
        bufs = (blk_a, blk_b)
        copies = [None, None]
        for g in range(_NCH):
            buf = bufs[g % 2]
            if copies[g % 2] is not None:
                copies[g % 2].wait()
            col0 = base + g * _CHUNK
            pltpu.async_copy(
                lt_hbm.at[tgt_v.at[pl.ds(g * _CHUNK, _CHUNK)],
                          pl.ds(col0, _CHUNK)],
                buf, sem_g).wait()
            copies[g % 2] = pltpu.async_copy(
                buf, blk_out.at[wid * _NCH + g], sem_o)
        for cp in copies:
            if cp is not None:
                cp.wait()

    return blocks_kernel(lt, targets)


def _sc_gather_w(weight_flat, indexes):
    mesh = plsc.VectorSubcoreMesh(core_axis_name="c", subcore_axis_name="s")

    @functools.partial(
        pl.kernel,
        mesh=mesh,
        out_type=jax.ShapeDtypeStruct((_B,), jnp.float32),
        scratch_types=[
            pltpu.VMEM((_PER_W,), jnp.int32),
            pltpu.VMEM((_PER_W,), jnp.float32),
            pltpu.SemaphoreType.DMA,
        ],
    )
    def gather_kernel(w_hbm, i_hbm, w_out, widx_v, w_v, sem):
        wid = lax.axis_index("c") * _NSUB + lax.axis_index("s")
        base = wid * _PER_W
        pltpu.sync_copy(i_hbm.at[pl.ds(base, _PER_W)], widx_v)
        copies = []
        for c in range(_NCH):
            sl = pl.ds(c * _CHUNK, _CHUNK)
            copies.append(pltpu.async_copy(
                w_hbm.at[widx_v.at[sl]], w_v.at[sl], sem))
        for cp in copies:
            cp.wait()
        pltpu.sync_copy(w_v, w_out.at[pl.ds(base, _PER_W)])

    return gather_kernel(weight_flat, indexes)


def _loss_body(blk_ref, w_ref, out_ref):
    blk = blk_ref[...]
    w = w_ref[...]
    shp = (_B // _CHUNK, _CHUNK, _CHUNK)
    bi = lax.broadcasted_iota(jnp.int32, shp, 1)
    li = lax.broadcasted_iota(jnp.int32, shp, 2)
    yg = jnp.sum(jnp.where(bi == li, blk, 0.0), axis=2)
    p = jnp.exp(jnp.log(yg) * _Q)
    out_ref[0, 0] = jnp.sum(((1.0 - p) * (1.0 / _Q) - _CONST) * w) * (1.0 / _B)


def kernel(logits, targets, indexes, weight):
    idx = indexes.astype(jnp.int32)
    tgt = targets.astype(jnp.int32)
    lt = logits.T
    blk = _sc_gather_blocks(lt, tgt)
    w = _sc_gather_w(weight.reshape(-1), idx)
    out = pl.pallas_call(
        _loss_body,
        out_shape=jax.ShapeDtypeStruct((1, 1), jnp.float32),
        out_specs=pl.BlockSpec(memory_space=pltpu.SMEM),
    )(blk, w.reshape(_B // _CHUNK, _CHUNK))
    return out[0, 0]


# speedup vs baseline: 2.7738x; 1.6617x over previous
import functools

import jax
import jax.numpy as jnp
from jax import lax
from jax.experimental import pallas as pl
from jax.experimental.pallas import tpu as pltpu
from jax.experimental.pallas import tpu_sc as plsc

_Q = 0.7
_K = 0.5
_B = 16384
_NCLS = 1000
_NCORES = 2
_NSUB = 16
_NW = _NCORES * _NSUB
_PER_W = _B // _NW
_CHUNK = 128
_NCH = _PER_W // _CHUNK
_NBLK = _B // _CHUNK
_GRID = 8
_CONST = (1.0 - _K ** _Q) / _Q


def _sc_gather_blocks(lt, targets):
    mesh = plsc.VectorSubcoreMesh(core_axis_name="c", subcore_axis_name="s")

    @functools.partial(
        pl.kernel,
        mesh=mesh,
        out_type=jax.ShapeDtypeStruct((_NBLK, _CHUNK, _CHUNK), jnp.float32),
        scratch_types=[
            pltpu.VMEM((_PER_W,), jnp.int32),
            pltpu.VMEM((_CHUNK, _CHUNK), jnp.float32),
            pltpu.VMEM((_CHUNK, _CHUNK), jnp.float32),
            pltpu.VMEM((_CHUNK, _CHUNK), jnp.float32),
            pltpu.VMEM((_CHUNK, _CHUNK), jnp.float32),
            pltpu.SemaphoreType.DMA,
            pltpu.SemaphoreType.DMA,
        ],
    )
    def blocks_kernel(lt_hbm, t_hbm, blk_out, tgt_v, b0, b1, b2, b3,
                      sem_g, sem_o):
        wid = lax.axis_index("c") * _NSUB + lax.axis_index("s")
        base = wid * _PER_W
        pltpu.sync_copy(t_hbm.at[pl.ds(base, _PER_W)], tgt_v)
        bufs = (b0, b1, b2, b3)
        gcopies = []
        for g in range(_NCH):
            gcopies.append(pltpu.async_copy(
                lt_hbm.at[tgt_v.at[pl.ds(g * _CHUNK, _CHUNK)],
                          pl.ds(base + g * _CHUNK, _CHUNK)],
                bufs[g], sem_g))
        ocopies = []
        for g in range(_NCH):
            gcopies[g].wait()
            ocopies.append(pltpu.async_copy(
                bufs[g], blk_out.at[wid * _NCH + g], sem_o))
        for cp in ocopies:
            cp.wait()

    return blocks_kernel(lt, targets)


def _sc_gather_w(weight_flat, indexes):
    mesh = plsc.VectorSubcoreMesh(core_axis_name="c", subcore_axis_name="s")

    @functools.partial(
        pl.kernel,
        mesh=mesh,
        out_type=jax.ShapeDtypeStruct((_B,), jnp.float32),
        scratch_types=[
            pltpu.VMEM((_PER_W,), jnp.int32),
            pltpu.VMEM((_PER_W,), jnp.float32),
            pltpu.SemaphoreType.DMA,
        ],
    )
    def gather_kernel(w_hbm, i_hbm, w_out, widx_v, w_v, sem):
        wid = lax.axis_index("c") * _NSUB + lax.axis_index("s")
        base = wid * _PER_W
        pltpu.sync_copy(i_hbm.at[pl.ds(base, _PER_W)], widx_v)
        copies = []
        for c in range(_NCH):
            sl = pl.ds(c * _CHUNK, _CHUNK)
            copies.append(pltpu.async_copy(
                w_hbm.at[widx_v.at[sl]], w_v.at[sl], sem))
        for cp in copies:
            cp.wait()
        pltpu.sync_copy(w_v, w_out.at[pl.ds(base, _PER_W)])

    return gather_kernel(weight_flat, indexes)


def _loss_body(blk_ref, w_ref, out_ref):
    i = pl.program_id(0)
    blk = blk_ref[...]
    w = w_ref[...]
    shp = (_NBLK // _GRID, _CHUNK, _CHUNK)
    bi = lax.broadcasted_iota(jnp.int32, shp, 1)
    li = lax.broadcasted_iota(jnp.int32, shp, 2)
    yg = jnp.sum(jnp.where(bi == li, blk, 0.0), axis=2)
    p = jnp.exp(jnp.log(yg) * _Q)
    part = jnp.sum(((1.0 - p) * (1.0 / _Q) - _CONST) * w) * (1.0 / _B)

    @pl.when(i == 0)
    def _():
        out_ref[0, 0] = 0.0

    out_ref[0, 0] += part


def kernel(logits, targets, indexes, weight):
    idx = indexes.astype(jnp.int32)
    tgt = targets.astype(jnp.int32)
    lt = logits.T
    blk = _sc_gather_blocks(lt, tgt)
    idx_dep, _ = jax.lax.optimization_barrier((idx, blk))
    w = _sc_gather_w(weight.reshape(-1), idx_dep)
    out = pl.pallas_call(
        _loss_body,
        grid=(_GRID,),
        in_specs=[
            pl.BlockSpec((_NBLK // _GRID, _CHUNK, _CHUNK), lambda i: (i, 0, 0)),
            pl.BlockSpec((_NBLK // _GRID, _CHUNK), lambda i: (i, 0)),
        ],
        out_specs=pl.BlockSpec((1, 1), lambda i: (0, 0),
                               memory_space=pltpu.SMEM),
        out_shape=jax.ShapeDtypeStruct((1, 1), jnp.float32),
    )(blk, w.reshape(_NBLK, _CHUNK))
    return out[0, 0]
